# baseline (speedup 1.0000x reference)
import functools

import jax
import jax.numpy as jnp
from jax import lax
from jax.experimental import pallas as pl
from jax.experimental.pallas import tpu as pltpu
from jax.experimental.pallas import tpu_sc as plsc

_CHUNK = 128
_KB = 16


@functools.cache
def _make_kernels(V, D, B):
    info = plsc.get_sparse_core_info()
    NC, NS = info.num_cores, info.num_subcores
    NW = NC * NS
    assert B % (8 * NW) == 0 and D % 8 == 0
    b_per_w = B // NW
    chunk = min(_CHUNK, b_per_w)
    n_chunks = b_per_w // chunk
    assert b_per_w % chunk == 0 and b_per_w % 16 == 0

    tiles_r = D // 8
    tiles_c = (V + 127) // 128
    full_c = V // 128
    v_full = full_c * 128
    n_tail = V - v_full
    n_tiles = tiles_r * tiles_c
    n_full = tiles_r * full_c
    t_per_w = (n_full + NW - 1) // NW
    n_batches = (t_per_w + _KB - 1) // _KB

    mesh = plsc.VectorSubcoreMesh(core_axis_name="c", subcore_axis_name="s")

    @functools.partial(
        pl.kernel,
        mesh=mesh,
        out_type=jax.ShapeDtypeStruct((n_tiles * 8, 128), jnp.float32),
        scratch_types=[
            pltpu.VMEM((8, 128 * _KB), jnp.float32),
            pltpu.SemaphoreType.DMA,
            pltpu.SemaphoreType.DMA,
        ],
        compiler_params=pltpu.CompilerParams(needs_layout_passes=False),
    )
    def detile_kernel(wt_hbm, q_hbm, buf, rsem, wsem):
        wid = lax.axis_index("s") * NC + lax.axis_index("c")
        lo = wid * t_per_w
        hi = jnp.minimum(lo + t_per_w, n_full)

        def refs(t):
            tr = t // full_c
            tc = t - tr * full_c
            src = wt_hbm.at[pl.ds(tr * 8, 8), pl.ds(tc * 128, 128)]
            dst = q_hbm.at[pl.ds((tr * tiles_c + tc) * 8, 8), :]
            return src, dst

        def batch(i, _):
            b0 = lo + i * _KB
            nb = hi - b0
            for k in range(_KB):
                @pl.when(k < nb)
                def _():
                    src, dst = refs(b0 + k)
                    pltpu.async_copy(src, dst, rsem)
            for k in range(_KB):
                @pl.when(k < nb)
                def _():
                    src, dst = refs(b0 + k)
                    pltpu.make_async_copy(src, dst, rsem).wait()
            return 0

        lax.fori_loop(0, n_batches, batch, 0)

    @functools.partial(
        pl.kernel,
        mesh=mesh,
        out_type=jax.ShapeDtypeStruct((D, B), jnp.float32),
        scratch_types=[
            pltpu.VMEM((b_per_w,), jnp.int32),
            pltpu.VMEM((D, b_per_w), jnp.int32),
            pltpu.VMEM((D, b_per_w), jnp.float32),
            pltpu.VMEM((n_tail * D,), jnp.float32),
            pltpu.SemaphoreType.DMA,
        ],
        compiler_params=pltpu.CompilerParams(
            use_tc_tiling_on_sc=False, needs_layout_passes=False
        ),
    )
    def gather_kernel(
        q_hbm, idx_hbm, tail_hbm, out_hbm, idx_v, offs_v, cols_v, tail_v, sem
    ):
        wid = lax.axis_index("s") * NC + lax.axis_index("c")
        base = wid * b_per_w
        pltpu.sync_copy(idx_hbm.at[pl.ds(base, b_per_w)], idx_v)
        pltpu.sync_copy(tail_hbm, tail_v)

        def xlate(g, _):
            c = idx_v[pl.ds(g * 16, 16)]
            tail = c >= v_full
            qoff = jnp.where(tail, 0, (c >> 7) * 1024 + (c & 127))
            for d in range(D):
                offs_v[d, pl.ds(g * 16, 16)] = qoff + (
                    (d // 8) * (tiles_c * 1024) + (d % 8) * 128
                )
            return 0

        lax.fori_loop(0, b_per_w // 16, xlate, 0)

        copies = [
            pltpu.async_copy(
                q_hbm.at[offs_v.at[d].at[pl.ds(j * chunk, chunk)]],
                cols_v.at[d].at[pl.ds(j * chunk, chunk)],
                sem,
            )
            for d in range(D)
            for j in range(n_chunks)
        ]
        for c in copies:
            c.wait()

        def patch(g, _):
            c = idx_v[pl.ds(g * 16, 16)]
            tail = c >= v_full
            toff = jnp.where(tail, (c - v_full) * D, 0)
            pos = lax.iota(jnp.int32, 16) + g * 16
            for d in range(D):
                val = plsc.load_gather(tail_v, [toff + d], mask=tail)
                plsc.store_scatter(
                    cols_v,
                    [jnp.full((16,), d, jnp.int32), pos],
                    val,
                    mask=tail,
                )
            return 0

        lax.fori_loop(0, b_per_w // 16, patch, 0)
        pltpu.sync_copy(cols_v, out_hbm.at[:, pl.ds(base, b_per_w)])

    return detile_kernel, gather_kernel, n_tiles, v_full


@jax.jit
def kernel(x, weight):
    (B,) = x.shape
    V, D = weight.shape
    detile, gather, n_tiles, v_full = _make_kernels(V, D, B)
    q = detile(weight.T)
    tail = weight[v_full:, :].reshape(-1)
    out_t = gather(q.reshape(n_tiles * 1024), x.astype(jnp.int32), tail)
    return out_t.T


# speedup vs baseline: 22.2091x; 22.2091x over previous
import functools

import jax
import jax.numpy as jnp
from jax import lax
from jax.experimental import pallas as pl
from jax.experimental.pallas import tpu as pltpu
from jax.experimental.pallas import tpu_sc as plsc

_CHUNK = 128
_KB = 64


@functools.cache
def _make_kernels(V, D, B):
    info = plsc.get_sparse_core_info()
    NC, NS = info.num_cores, info.num_subcores
    NW = NC * NS
    assert B % (8 * NW) == 0 and D % 8 == 0
    b_per_w = B // NW
    chunk = min(_CHUNK, b_per_w)
    n_chunks = b_per_w // chunk
    assert b_per_w % chunk == 0 and b_per_w % 16 == 0

    tiles_r = D // 8
    tiles_c = (V + 127) // 128
    full_c = V // 128
    v_full = full_c * 128
    n_tail = V - v_full
    n_tiles = tiles_r * tiles_c
    n_full = tiles_r * full_c
    t_per_w = (n_full + NW - 1) // NW
    n_batches = (t_per_w + _KB - 1) // _KB

    mesh = plsc.VectorSubcoreMesh(core_axis_name="c", subcore_axis_name="s")

    @functools.partial(
        pl.kernel,
        mesh=mesh,
        out_type=jax.ShapeDtypeStruct((n_tiles * 8, 128), jnp.float32),
        scratch_types=[
            pltpu.VMEM((8, 128 * _KB), jnp.float32),
            pltpu.SemaphoreType.DMA,
            pltpu.SemaphoreType.DMA,
        ],
        compiler_params=pltpu.CompilerParams(needs_layout_passes=False),
    )
    def detile_kernel(wt_hbm, q_hbm, buf, rsem, wsem):
        wid = lax.axis_index("s") * NC + lax.axis_index("c")
        lo = wid * t_per_w
        hi = jnp.minimum(lo + t_per_w, n_full)

        def refs(t):
            tr = t // full_c
            tc = t - tr * full_c
            src = wt_hbm.at[pl.ds(tr * 8, 8), pl.ds(tc * 128, 128)]
            dst = q_hbm.at[pl.ds((tr * tiles_c + tc) * 8, 8), :]
            return src, dst

        def batch(i, _):
            b0 = lo + i * _KB
            nb = hi - b0
            for k in range(_KB):
                @pl.when(k < nb)
                def _():
                    src, _dst = refs(b0 + k)
                    pltpu.async_copy(src, buf.at[:, pl.ds(k * 128, 128)], rsem)
            for k in range(_KB):
                @pl.when(k < nb)
                def _():
                    src, dst = refs(b0 + k)
                    pltpu.make_async_copy(
                        src, buf.at[:, pl.ds(k * 128, 128)], rsem
                    ).wait()
                    pltpu.async_copy(buf.at[:, pl.ds(k * 128, 128)], dst, wsem)
            for k in range(_KB):
                @pl.when(k < nb)
                def _():
                    _src, dst = refs(b0 + k)
                    pltpu.make_async_copy(
                        buf.at[:, pl.ds(k * 128, 128)], dst, wsem
                    ).wait()
            return 0

        lax.fori_loop(0, n_batches, batch, 0)

    @functools.partial(
        pl.kernel,
        mesh=mesh,
        out_type=jax.ShapeDtypeStruct((D, B), jnp.float32),
        scratch_types=[
            pltpu.VMEM((b_per_w,), jnp.int32),
            pltpu.VMEM((D, b_per_w), jnp.int32),
            pltpu.VMEM((D, b_per_w), jnp.float32),
            pltpu.VMEM((n_tail * D,), jnp.float32),
            pltpu.SemaphoreType.DMA,
        ],
        compiler_params=pltpu.CompilerParams(
            use_tc_tiling_on_sc=False, needs_layout_passes=False
        ),
    )
    def gather_kernel(
        q_hbm, idx_hbm, tail_hbm, out_hbm, idx_v, offs_v, cols_v, tail_v, sem
    ):
        wid = lax.axis_index("s") * NC + lax.axis_index("c")
        base = wid * b_per_w
        pltpu.sync_copy(idx_hbm.at[pl.ds(base, b_per_w)], idx_v)
        pltpu.sync_copy(tail_hbm, tail_v)

        def xlate(g, _):
            c = idx_v[pl.ds(g * 16, 16)]
            tail = c >= v_full
            qoff = jnp.where(tail, 0, (c >> 7) * 1024 + (c & 127))
            for d in range(D):
                offs_v[d, pl.ds(g * 16, 16)] = qoff + (
                    (d // 8) * (tiles_c * 1024) + (d % 8) * 128
                )
            return 0

        lax.fori_loop(0, b_per_w // 16, xlate, 0)

        copies = [
            pltpu.async_copy(
                q_hbm.at[offs_v.at[d].at[pl.ds(j * chunk, chunk)]],
                cols_v.at[d].at[pl.ds(j * chunk, chunk)],
                sem,
            )
            for d in range(D)
            for j in range(n_chunks)
        ]
        for c in copies:
            c.wait()

        def patch(g, _):
            c = idx_v[pl.ds(g * 16, 16)]
            tail = c >= v_full
            toff = jnp.where(tail, (c - v_full) * D, 0)
            pos = lax.iota(jnp.int32, 16) + g * 16
            for d in range(D):
                val = plsc.load_gather(tail_v, [toff + d], mask=tail)
                plsc.store_scatter(
                    cols_v,
                    [jnp.full((16,), d, jnp.int32), pos],
                    val,
                    mask=tail,
                )
            return 0

        lax.fori_loop(0, b_per_w // 16, patch, 0)
        pltpu.sync_copy(cols_v, out_hbm.at[:, pl.ds(base, b_per_w)])

    return detile_kernel, gather_kernel, n_tiles, v_full


@jax.jit
def kernel(x, weight):
    (B,) = x.shape
    V, D = weight.shape
    detile, gather, n_tiles, v_full = _make_kernels(V, D, B)
    q = detile(weight.T)
    tail = weight[v_full:, :].reshape(-1)
    out_t = gather(q.reshape(n_tiles * 1024), x.astype(jnp.int32), tail)
    return out_t.T


# speedup vs baseline: 23.3875x; 1.0531x over previous
import functools

import jax
import jax.numpy as jnp
from jax import lax
from jax.experimental import pallas as pl
from jax.experimental.pallas import tpu as pltpu
from jax.experimental.pallas import tpu_sc as plsc

_CHUNK = 128
_KB = 16


@functools.cache
def _make_kernels(V, D, B):
    info = plsc.get_sparse_core_info()
    NC, NS = info.num_cores, info.num_subcores
    NW = NC * NS
    assert B % (8 * NW) == 0 and D % 8 == 0
    b_per_w = B // NW
    chunk = min(_CHUNK, b_per_w)
    n_chunks = b_per_w // chunk
    assert b_per_w % chunk == 0 and b_per_w % 16 == 0

    tiles_r = D // 8
    tiles_c = (V + 127) // 128
    full_c = V // 128
    v_full = full_c * 128
    n_tail = V - v_full
    n_tiles = tiles_r * tiles_c
    n_full = tiles_r * full_c
    t_per_w = (n_full + NW - 1) // NW
    n_batches = (t_per_w + _KB - 1) // _KB

    mesh = plsc.VectorSubcoreMesh(core_axis_name="c", subcore_axis_name="s")

    @functools.partial(
        pl.kernel,
        mesh=mesh,
        out_type=jax.ShapeDtypeStruct((n_tiles * 8, 128), jnp.float32),
        scratch_types=[
            pltpu.VMEM((8, 128 * _KB), jnp.float32),
            pltpu.SemaphoreType.DMA,
            pltpu.SemaphoreType.DMA,
        ],
        compiler_params=pltpu.CompilerParams(needs_layout_passes=False),
    )
    def detile_kernel(wt_hbm, q_hbm, buf, rsem, wsem):
        wid = lax.axis_index("s") * NC + lax.axis_index("c")
        lo = wid * t_per_w
        hi = jnp.minimum(lo + t_per_w, n_full)

        def refs(t):
            tr = t // full_c
            tc = t - tr * full_c
            src = wt_hbm.at[pl.ds(tr * 8, 8), pl.ds(tc * 128, 128)]
            dst = q_hbm.at[pl.ds((tr * tiles_c + tc) * 8, 8), :]
            return src, dst

        def batch(i, _):
            b0 = lo + i * _KB
            nb = hi - b0
            for k in range(_KB):
                @pl.when(k < nb)
                def _():
                    src, _dst = refs(b0 + k)
                    pltpu.async_copy(src, buf.at[:, pl.ds(k * 128, 128)], rsem)
            for k in range(_KB):
                @pl.when(k < nb)
                def _():
                    src, dst = refs(b0 + k)
                    pltpu.make_async_copy(
                        src, buf.at[:, pl.ds(k * 128, 128)], rsem
                    ).wait()
                    pltpu.async_copy(buf.at[:, pl.ds(k * 128, 128)], dst, wsem)
            for k in range(_KB):
                @pl.when(k < nb)
                def _():
                    _src, dst = refs(b0 + k)
                    pltpu.make_async_copy(
                        buf.at[:, pl.ds(k * 128, 128)], dst, wsem
                    ).wait()
            return 0

        lax.fori_loop(0, n_batches, batch, 0)

    @functools.partial(
        pl.kernel,
        mesh=mesh,
        out_type=jax.ShapeDtypeStruct((D, B), jnp.float32),
        scratch_types=[
            pltpu.VMEM((b_per_w,), jnp.int32),
            pltpu.VMEM((D, b_per_w), jnp.int32),
            pltpu.VMEM((D, b_per_w), jnp.float32),
            pltpu.VMEM((n_tail * D,), jnp.float32),
            pltpu.SemaphoreType.DMA,
        ],
        compiler_params=pltpu.CompilerParams(
            use_tc_tiling_on_sc=False, needs_layout_passes=False
        ),
    )
    def gather_kernel(
        q_hbm, idx_hbm, tail_hbm, out_hbm, idx_v, offs_v, cols_v, tail_v, sem
    ):
        wid = lax.axis_index("s") * NC + lax.axis_index("c")
        base = wid * b_per_w
        pltpu.sync_copy(idx_hbm.at[pl.ds(base, b_per_w)], idx_v)
        pltpu.sync_copy(tail_hbm, tail_v)

        def xlate(g, _):
            c = idx_v[pl.ds(g * 16, 16)]
            tail = c >= v_full
            qoff = jnp.where(tail, 0, (c >> 7) * 1024 + (c & 127))
            for d in range(D):
                offs_v[d, pl.ds(g * 16, 16)] = qoff + (
                    (d // 8) * (tiles_c * 1024) + (d % 8) * 128
                )
            return 0

        lax.fori_loop(0, b_per_w // 16, xlate, 0)

        copies = [
            pltpu.async_copy(
                q_hbm.at[offs_v.at[d].at[pl.ds(j * chunk, chunk)]],
                cols_v.at[d].at[pl.ds(j * chunk, chunk)],
                sem,
            )
            for d in range(D)
            for j in range(n_chunks)
        ]
        for c in copies:
            c.wait()

        def patch(g, _):
            c = idx_v[pl.ds(g * 16, 16)]
            tail = c >= v_full
            toff = jnp.where(tail, (c - v_full) * D, 0)
            pos = lax.iota(jnp.int32, 16) + g * 16
            for d in range(D):
                val = plsc.load_gather(tail_v, [toff + d], mask=tail)
                plsc.store_scatter(
                    cols_v,
                    [jnp.full((16,), d, jnp.int32), pos],
                    val,
                    mask=tail,
                )
            return 0

        lax.fori_loop(0, b_per_w // 16, patch, 0)
        pltpu.sync_copy(cols_v, out_hbm.at[:, pl.ds(base, b_per_w)])

    return detile_kernel, gather_kernel, n_tiles, v_full


@jax.jit
def kernel(x, weight):
    (B,) = x.shape
    V, D = weight.shape
    detile, gather, n_tiles, v_full = _make_kernels(V, D, B)
    q = detile(weight.T)
    tail = weight[v_full:, :].reshape(-1)
    out_t = gather(q.reshape(n_tiles * 1024), x.astype(jnp.int32), tail)
    return out_t.T
